# initial kernel scaffold (unmeasured)
import jax
import jax.numpy as jnp
from jax import lax
from jax.experimental import pallas as pl
from jax.experimental.pallas import tpu as pltpu

N_DEV = 16
M = 4096
N = 2048
CHUNK = M // N_DEV


def kernel(x, w_mat):
    k_per = x.shape[1]
    assert x.shape == (M, k_per), x.shape
    assert w_mat.shape == (k_per, N), w_mat.shape

    def body(x_ref, w_ref, out_ref, comm_ref, send_sems, recv_sems):
        my = lax.axis_index("i")
        left = (my + N_DEV - 1) % N_DEV
        right = (my + 1) % N_DEV

        barrier_sem = pltpu.get_barrier_semaphore()
        for nbr in (left, right):
            pl.semaphore_signal(
                barrier_sem, inc=1,
                device_id=(nbr,), device_id_type=pl.DeviceIdType.MESH,
            )
        pl.semaphore_wait(barrier_sem, 2)

        def partial_chunk(c):
            xr = x_ref[pl.ds(c * CHUNK, CHUNK), :]
            return jnp.dot(xr, w_ref[...], preferred_element_type=jnp.float32)

        comm_ref[0] = partial_chunk(my)
        for s in range(N_DEV - 1):
            send_slot = s % 2
            recv_slot = (s + 1) % 2
            rdma = pltpu.make_async_remote_copy(
                src_ref=comm_ref.at[send_slot],
                dst_ref=comm_ref.at[recv_slot],
                send_sem=send_sems.at[send_slot],
                recv_sem=recv_sems.at[recv_slot],
                device_id=(right,),
                device_id_type=pl.DeviceIdType.MESH,
            )
            rdma.start()
            c = (my + N_DEV - 1 - s) % N_DEV
            p = partial_chunk(c)
            rdma.wait()
            comm_ref[recv_slot] = comm_ref[recv_slot] + p

        own = (my + 1) % N_DEV
        out_ref[pl.ds(own * CHUNK, CHUNK), :] = comm_ref[1]
        amax = jnp.max(jnp.abs(comm_ref[1]))

        for t in range(N_DEV - 1):
            h = (N_DEV - 1) + t
            send_slot = h % 2
            recv_slot = (h + 1) % 2
            rdma = pltpu.make_async_remote_copy(
                src_ref=comm_ref.at[send_slot],
                dst_ref=comm_ref.at[recv_slot],
                send_sem=send_sems.at[send_slot],
                recv_sem=recv_sems.at[recv_slot],
                device_id=(right,),
                device_id_type=pl.DeviceIdType.MESH,
            )
            rdma.start()
            rdma.wait()
            c = (my + N_DEV - t) % N_DEV
            out_ref[pl.ds(c * CHUNK, CHUNK), :] = comm_ref[recv_slot]
            amax = jnp.maximum(amax, jnp.max(jnp.abs(comm_ref[recv_slot])))

        scale = amax / 448.0
        for i in range(N_DEV):
            v = out_ref[pl.ds(i * CHUNK, CHUNK), :]
            q = (v / scale).astype(jnp.float8_e4m3fn)
            out_ref[pl.ds(i * CHUNK, CHUNK), :] = q.astype(jnp.float32) * scale

    return pl.pallas_call(
        body,
        out_shape=jax.ShapeDtypeStruct((M, N), jnp.float32),
        in_specs=[
            pl.BlockSpec(memory_space=pltpu.VMEM),
            pl.BlockSpec(memory_space=pltpu.VMEM),
        ],
        out_specs=pl.BlockSpec(memory_space=pltpu.VMEM),
        scratch_shapes=[
            pltpu.VMEM((2, CHUNK, N), jnp.float32),
            pltpu.SemaphoreType.DMA((2,)),
            pltpu.SemaphoreType.DMA((2,)),
        ],
        compiler_params=pltpu.CompilerParams(collective_id=7),
    )(x, w_mat)


# baseline (device time: 783785 ns/iter reference)
import jax
import jax.numpy as jnp
from jax import lax
from jax.experimental import pallas as pl
from jax.experimental.pallas import tpu as pltpu

N_DEV = 16
M = 4096
N = 2048
CHUNK = M // N_DEV


def kernel(x, w_mat):
    k_per = x.shape[1]
    assert x.shape == (M, k_per), x.shape
    assert w_mat.shape == (k_per, N), w_mat.shape

    def body(x_ref, w_ref, out_ref, comm_ref, send_sems, recv_sems):
        my = lax.axis_index("i")
        left = (my + N_DEV - 1) % N_DEV
        right = (my + 1) % N_DEV

        barrier_sem = pltpu.get_barrier_semaphore()
        for nbr in (left, right):
            pl.semaphore_signal(
                barrier_sem, inc=1,
                device_id=(nbr,), device_id_type=pl.DeviceIdType.MESH,
            )
        pl.semaphore_wait(barrier_sem, 2)

        def partial_chunk(c):
            xr = x_ref[pl.ds(c * CHUNK, CHUNK), :]
            return jnp.dot(xr, w_ref[...], preferred_element_type=jnp.float32)

        comm_ref[0] = partial_chunk(my)
        for s in range(N_DEV - 1):
            send_slot = s % 2
            recv_slot = (s + 1) % 2
            rdma = pltpu.make_async_remote_copy(
                src_ref=comm_ref.at[send_slot],
                dst_ref=comm_ref.at[recv_slot],
                send_sem=send_sems.at[send_slot],
                recv_sem=recv_sems.at[recv_slot],
                device_id=(right,),
                device_id_type=pl.DeviceIdType.MESH,
            )
            rdma.start()
            c = (my + N_DEV - 1 - s) % N_DEV
            p = partial_chunk(c)
            rdma.wait()
            comm_ref[recv_slot] = comm_ref[recv_slot] + p

        own = (my + 1) % N_DEV
        out_ref[pl.ds(own * CHUNK, CHUNK), :] = comm_ref[1]
        amax = jnp.max(jnp.abs(comm_ref[1]))

        for t in range(N_DEV - 1):
            h = (N_DEV - 1) + t
            send_slot = h % 2
            recv_slot = (h + 1) % 2
            rdma = pltpu.make_async_remote_copy(
                src_ref=comm_ref.at[send_slot],
                dst_ref=comm_ref.at[recv_slot],
                send_sem=send_sems.at[send_slot],
                recv_sem=recv_sems.at[recv_slot],
                device_id=(right,),
                device_id_type=pl.DeviceIdType.MESH,
            )
            rdma.start()
            rdma.wait()
            c = (my + N_DEV - t) % N_DEV
            out_ref[pl.ds(c * CHUNK, CHUNK), :] = comm_ref[recv_slot]
            amax = jnp.maximum(amax, jnp.max(jnp.abs(comm_ref[recv_slot])))

        scale = amax / 448.0
        for i in range(N_DEV):
            v = out_ref[pl.ds(i * CHUNK, CHUNK), :]
            q = (v / scale).astype(jnp.float8_e4m3fn)
            out_ref[pl.ds(i * CHUNK, CHUNK), :] = q.astype(jnp.float32) * scale

    return pl.pallas_call(
        body,
        out_shape=jax.ShapeDtypeStruct((M, N), jnp.float32),
        in_specs=[
            pl.BlockSpec(memory_space=pltpu.VMEM),
            pl.BlockSpec(memory_space=pltpu.VMEM),
        ],
        out_specs=pl.BlockSpec(memory_space=pltpu.VMEM),
        scratch_shapes=[
            pltpu.VMEM((2, CHUNK, N), jnp.float32),
            pltpu.SemaphoreType.DMA((2,)),
            pltpu.SemaphoreType.DMA((2,)),
        ],
        compiler_params=pltpu.CompilerParams(
            collective_id=7, vmem_limit_bytes=100 * 1024 * 1024
        ),
    )(x, w_mat)


# device time: 357275 ns/iter; 2.1938x vs baseline; 2.1938x over previous
import jax
import jax.numpy as jnp
from jax import lax
from jax.experimental import pallas as pl
from jax.experimental.pallas import tpu as pltpu

N_DEV = 16
M = 4096
N = 2048
CHUNK = M // N_DEV
HALF = N // 2

F8 = jnp.float8_e4m3fn


def kernel(x, w_mat):
    k_per = x.shape[1]
    assert x.shape == (M, k_per), x.shape
    assert w_mat.shape == (k_per, N), w_mat.shape

    def body(x_ref, w_ref, out_ref, xb_ref, wb_ref,
             comm_r, comm_l, send_r, recv_r, send_l, recv_l,
             q_r, q_l, qsend_r, qrecv_r, qsend_l, qrecv_l,
             amax_buf, asend, arecv):
        my = lax.axis_index("i")
        left = (my + N_DEV - 1) % N_DEV
        right = (my + 1) % N_DEV

        barrier_sem = pltpu.get_barrier_semaphore()
        for nbr in (left, right):
            pl.semaphore_signal(
                barrier_sem, inc=1,
                device_id=(nbr,), device_id_type=pl.DeviceIdType.MESH,
            )
        pl.semaphore_wait(barrier_sem, 2)

        xb_ref[...] = x_ref[...].astype(jnp.bfloat16)
        wb_ref[...] = w_ref[...].astype(jnp.bfloat16)

        def partial_half(c, lo):
            xr = xb_ref[pl.ds(c * CHUNK, CHUNK), :]
            return jnp.dot(xr, wb_ref[:, lo:lo + HALF],
                           preferred_element_type=jnp.float32)

        comm_r[0] = partial_half(my, 0)
        comm_l[0] = partial_half(my, HALF)
        for s in range(N_DEV - 1):
            ss, rs = s % 2, (s + 1) % 2
            r = pltpu.make_async_remote_copy(
                src_ref=comm_r.at[ss], dst_ref=comm_r.at[rs],
                send_sem=send_r.at[ss], recv_sem=recv_r.at[rs],
                device_id=(right,), device_id_type=pl.DeviceIdType.MESH,
            )
            l = pltpu.make_async_remote_copy(
                src_ref=comm_l.at[ss], dst_ref=comm_l.at[rs],
                send_sem=send_l.at[ss], recv_sem=recv_l.at[rs],
                device_id=(left,), device_id_type=pl.DeviceIdType.MESH,
            )
            r.start()
            l.start()
            c_r = (my + N_DEV - 1 - s) % N_DEV
            c_l = (my + s + 1) % N_DEV
            p_r = partial_half(c_r, 0)
            p_l = partial_half(c_l, HALF)
            r.wait()
            l.wait()
            comm_r[rs] = comm_r[rs] + p_r
            comm_l[rs] = comm_l[rs] + p_l

        own_r = (my + 1) % N_DEV
        own_l = (my + N_DEV - 1) % N_DEV

        local_amax = jnp.maximum(jnp.max(jnp.abs(comm_r[1])),
                                 jnp.max(jnp.abs(comm_l[1])))
        amax_buf[my] = jnp.full((8, 128), local_amax, jnp.float32)
        sends = []
        for o in range(1, N_DEV):
            tgt = (my + o) % N_DEV
            rd = pltpu.make_async_remote_copy(
                src_ref=amax_buf.at[my], dst_ref=amax_buf.at[my],
                send_sem=asend.at[o], recv_sem=arecv.at[my],
                device_id=(tgt,), device_id_type=pl.DeviceIdType.MESH,
            )
            rd.start()
            sends.append(rd)
        for o in range(1, N_DEV):
            src = (my + o) % N_DEV
            rd = pltpu.make_async_remote_copy(
                src_ref=amax_buf.at[src], dst_ref=amax_buf.at[src],
                send_sem=asend.at[o], recv_sem=arecv.at[src],
                device_id=(src,), device_id_type=pl.DeviceIdType.MESH,
            )
            rd.wait_recv()
        for rd in sends:
            rd.wait_send()
        amax = jnp.max(amax_buf[:, 0, 0])
        scale = amax / 448.0
        inv_scale = 1.0 / scale

        q_r[0] = (comm_r[1] * inv_scale).astype(F8)
        q_l[0] = (comm_l[1] * inv_scale).astype(F8)
        out_ref[pl.ds(own_r * CHUNK, CHUNK), 0:HALF] = (
            q_r[0].astype(jnp.float32) * scale)
        out_ref[pl.ds(own_l * CHUNK, CHUNK), HALF:N] = (
            q_l[0].astype(jnp.float32) * scale)
        for t in range(N_DEV - 1):
            ss, rs = t % 2, (t + 1) % 2
            r = pltpu.make_async_remote_copy(
                src_ref=q_r.at[ss], dst_ref=q_r.at[rs],
                send_sem=qsend_r.at[ss], recv_sem=qrecv_r.at[rs],
                device_id=(right,), device_id_type=pl.DeviceIdType.MESH,
            )
            l = pltpu.make_async_remote_copy(
                src_ref=q_l.at[ss], dst_ref=q_l.at[rs],
                send_sem=qsend_l.at[ss], recv_sem=qrecv_l.at[rs],
                device_id=(left,), device_id_type=pl.DeviceIdType.MESH,
            )
            r.start()
            l.start()
            r.wait()
            l.wait()
            c_r = (my + N_DEV - t) % N_DEV
            c_l = (my + t) % N_DEV
            out_ref[pl.ds(c_r * CHUNK, CHUNK), 0:HALF] = (
                q_r[rs].astype(jnp.float32) * scale)
            out_ref[pl.ds(c_l * CHUNK, CHUNK), HALF:N] = (
                q_l[rs].astype(jnp.float32) * scale)

    return pl.pallas_call(
        body,
        out_shape=jax.ShapeDtypeStruct((M, N), jnp.float32),
        in_specs=[
            pl.BlockSpec(memory_space=pltpu.VMEM),
            pl.BlockSpec(memory_space=pltpu.VMEM),
        ],
        out_specs=pl.BlockSpec(memory_space=pltpu.VMEM),
        scratch_shapes=[
            pltpu.VMEM((M, k_per), jnp.bfloat16),
            pltpu.VMEM((k_per, N), jnp.bfloat16),
            pltpu.VMEM((2, CHUNK, HALF), jnp.float32),
            pltpu.VMEM((2, CHUNK, HALF), jnp.float32),
            pltpu.SemaphoreType.DMA((2,)),
            pltpu.SemaphoreType.DMA((2,)),
            pltpu.SemaphoreType.DMA((2,)),
            pltpu.SemaphoreType.DMA((2,)),
            pltpu.VMEM((2, CHUNK, HALF), F8),
            pltpu.VMEM((2, CHUNK, HALF), F8),
            pltpu.SemaphoreType.DMA((2,)),
            pltpu.SemaphoreType.DMA((2,)),
            pltpu.SemaphoreType.DMA((2,)),
            pltpu.SemaphoreType.DMA((2,)),
            pltpu.VMEM((N_DEV, 8, 128), jnp.float32),
            pltpu.SemaphoreType.DMA((N_DEV,)),
            pltpu.SemaphoreType.DMA((N_DEV,)),
        ],
        compiler_params=pltpu.CompilerParams(
            collective_id=7, vmem_limit_bytes=100 * 1024 * 1024
        ),
    )(x, w_mat)


# device time: 307977 ns/iter; 2.5449x vs baseline; 1.1601x over previous
import jax
import jax.numpy as jnp
from jax import lax
from jax.experimental import pallas as pl
from jax.experimental.pallas import tpu as pltpu

N_DEV = 16
M = 4096
N = 2048
CHUNK = M // N_DEV
HALF = N // 2

F8 = jnp.float8_e4m3fn


def kernel(x, w_mat):
    k_per = x.shape[1]
    assert x.shape == (M, k_per), x.shape
    assert w_mat.shape == (k_per, N), w_mat.shape

    def body(x_ref, w_ref, out_ref, xb_ref, wb_ref,
             comm_r, comm_l, comm_rb, comm_lb,
             send_r, recv_r, send_l, recv_l,
             q_r, q_l, qsend_r, qrecv_r, qsend_l, qrecv_l,
             amax_buf, asend, arecv):
        my = lax.axis_index("i")
        left = (my + N_DEV - 1) % N_DEV
        right = (my + 1) % N_DEV

        barrier_sem = pltpu.get_barrier_semaphore()
        for nbr in (left, right):
            pl.semaphore_signal(
                barrier_sem, inc=1,
                device_id=(nbr,), device_id_type=pl.DeviceIdType.MESH,
            )
        pl.semaphore_wait(barrier_sem, 2)

        xb_ref[...] = x_ref[...].astype(jnp.bfloat16)
        wb_ref[...] = w_ref[...].astype(jnp.bfloat16)

        def partial_half(c, lo):
            xr = xb_ref[pl.ds(c * CHUNK, CHUNK), :]
            return jnp.dot(xr, wb_ref[:, lo:lo + HALF],
                           preferred_element_type=jnp.float32)

        NB = 8

        def rs_hop(s, src_r, src_l, dst_r, dst_l):
            ss, rs = s % 2, (s + 1) % 2
            r = pltpu.make_async_remote_copy(
                src_ref=src_r.at[ss], dst_ref=dst_r.at[rs],
                send_sem=send_r.at[ss], recv_sem=recv_r.at[rs],
                device_id=(right,), device_id_type=pl.DeviceIdType.MESH,
            )
            l = pltpu.make_async_remote_copy(
                src_ref=src_l.at[ss], dst_ref=dst_l.at[rs],
                send_sem=send_l.at[ss], recv_sem=recv_l.at[rs],
                device_id=(left,), device_id_type=pl.DeviceIdType.MESH,
            )
            r.start()
            l.start()
            return r, l, rs

        comm_rb[0] = partial_half(my, 0).astype(jnp.bfloat16)
        comm_lb[0] = partial_half(my, HALF).astype(jnp.bfloat16)
        for s in range(N_DEV - 1):
            bf_send = s < NB
            bf_next = (s + 1) < NB
            sr = comm_rb if bf_send else comm_r
            sl = comm_lb if bf_send else comm_l
            r, l, rs = rs_hop(s, sr, sl, sr, sl)
            c_r = (my + N_DEV - 1 - s) % N_DEV
            c_l = (my + s + 1) % N_DEV
            p_r = partial_half(c_r, 0)
            p_l = partial_half(c_l, HALF)
            r.wait()
            acc_r = (sr[rs].astype(jnp.float32) if bf_send else sr[rs]) + p_r
            if bf_next:
                comm_rb[rs] = acc_r.astype(jnp.bfloat16)
            else:
                comm_r[rs] = acc_r
            l.wait()
            acc_l = (sl[rs].astype(jnp.float32) if bf_send else sl[rs]) + p_l
            if bf_next:
                comm_lb[rs] = acc_l.astype(jnp.bfloat16)
            else:
                comm_l[rs] = acc_l

        own_r = (my + 1) % N_DEV
        own_l = (my + N_DEV - 1) % N_DEV

        local_amax = jnp.maximum(jnp.max(jnp.abs(comm_r[1])),
                                 jnp.max(jnp.abs(comm_l[1])))
        amax_buf[my] = jnp.full((8, 128), local_amax, jnp.float32)
        sends = []
        for o in range(1, N_DEV):
            tgt = (my + o) % N_DEV
            rd = pltpu.make_async_remote_copy(
                src_ref=amax_buf.at[my], dst_ref=amax_buf.at[my],
                send_sem=asend.at[o], recv_sem=arecv.at[my],
                device_id=(tgt,), device_id_type=pl.DeviceIdType.MESH,
            )
            rd.start()
            sends.append(rd)
        for o in range(1, N_DEV):
            src = (my + o) % N_DEV
            rd = pltpu.make_async_remote_copy(
                src_ref=amax_buf.at[src], dst_ref=amax_buf.at[src],
                send_sem=asend.at[o], recv_sem=arecv.at[src],
                device_id=(src,), device_id_type=pl.DeviceIdType.MESH,
            )
            rd.wait_recv()
        for rd in sends:
            rd.wait_send()
        amax = jnp.max(amax_buf[:, 0, 0])
        scale = amax / 448.0
        inv_scale = 1.0 / scale

        def ag_hop(t):
            ss, rs = t % 2, (t + 1) % 2
            r = pltpu.make_async_remote_copy(
                src_ref=q_r.at[ss], dst_ref=q_r.at[rs],
                send_sem=qsend_r.at[ss], recv_sem=qrecv_r.at[rs],
                device_id=(right,), device_id_type=pl.DeviceIdType.MESH,
            )
            l = pltpu.make_async_remote_copy(
                src_ref=q_l.at[ss], dst_ref=q_l.at[rs],
                send_sem=qsend_l.at[ss], recv_sem=qrecv_l.at[rs],
                device_id=(left,), device_id_type=pl.DeviceIdType.MESH,
            )
            r.start()
            l.start()
            return r, l

        q_r[0] = (comm_r[1] * inv_scale).astype(F8)
        q_l[0] = (comm_l[1] * inv_scale).astype(F8)
        pend = ag_hop(0)
        out_ref[pl.ds(own_r * CHUNK, CHUNK), 0:HALF] = (
            q_r[0].astype(jnp.float32) * scale)
        out_ref[pl.ds(own_l * CHUNK, CHUNK), HALF:N] = (
            q_l[0].astype(jnp.float32) * scale)
        for t in range(N_DEV - 1):
            rs = (t + 1) % 2
            pend[0].wait()
            pend[1].wait()
            if t < N_DEV - 2:
                pend = ag_hop(t + 1)
            c_r = (my + N_DEV - t) % N_DEV
            c_l = (my + t) % N_DEV
            out_ref[pl.ds(c_r * CHUNK, CHUNK), 0:HALF] = (
                q_r[rs].astype(jnp.float32) * scale)
            out_ref[pl.ds(c_l * CHUNK, CHUNK), HALF:N] = (
                q_l[rs].astype(jnp.float32) * scale)

    return pl.pallas_call(
        body,
        out_shape=jax.ShapeDtypeStruct((M, N), jnp.float32),
        in_specs=[
            pl.BlockSpec(memory_space=pltpu.VMEM),
            pl.BlockSpec(memory_space=pltpu.VMEM),
        ],
        out_specs=pl.BlockSpec(memory_space=pltpu.VMEM),
        scratch_shapes=[
            pltpu.VMEM((M, k_per), jnp.bfloat16),
            pltpu.VMEM((k_per, N), jnp.bfloat16),
            pltpu.VMEM((2, CHUNK, HALF), jnp.float32),
            pltpu.VMEM((2, CHUNK, HALF), jnp.float32),
            pltpu.VMEM((2, CHUNK, HALF), jnp.bfloat16),
            pltpu.VMEM((2, CHUNK, HALF), jnp.bfloat16),
            pltpu.SemaphoreType.DMA((2,)),
            pltpu.SemaphoreType.DMA((2,)),
            pltpu.SemaphoreType.DMA((2,)),
            pltpu.SemaphoreType.DMA((2,)),
            pltpu.VMEM((2, CHUNK, HALF), F8),
            pltpu.VMEM((2, CHUNK, HALF), F8),
            pltpu.SemaphoreType.DMA((2,)),
            pltpu.SemaphoreType.DMA((2,)),
            pltpu.SemaphoreType.DMA((2,)),
            pltpu.SemaphoreType.DMA((2,)),
            pltpu.VMEM((N_DEV, 8, 128), jnp.float32),
            pltpu.SemaphoreType.DMA((N_DEV,)),
            pltpu.SemaphoreType.DMA((N_DEV,)),
        ],
        compiler_params=pltpu.CompilerParams(
            collective_id=7, vmem_limit_bytes=100 * 1024 * 1024
        ),
    )(x, w_mat)


# device time: 262271 ns/iter; 2.9885x vs baseline; 1.1743x over previous
import jax
import jax.numpy as jnp
from jax import lax
from jax.experimental import pallas as pl
from jax.experimental.pallas import tpu as pltpu

N_DEV = 16
M = 4096
N = 2048
CHUNK = M // N_DEV
HALF = N // 2

F8 = jnp.float8_e4m3fn


def kernel(x, w_mat):
    k_per = x.shape[1]
    assert x.shape == (M, k_per), x.shape
    assert w_mat.shape == (k_per, N), w_mat.shape

    def body(x_ref, w_ref, out_ref, xb_ref, wb_ref,
             comm_r, comm_l, comm_rb, comm_lb,
             send_r, recv_r, send_l, recv_l,
             q_r, q_l, qsend_r, qrecv_r, qsend_l, qrecv_l,
             amax_buf, asend, arecv):
        my = lax.axis_index("i")
        left = (my + N_DEV - 1) % N_DEV
        right = (my + 1) % N_DEV

        barrier_sem = pltpu.get_barrier_semaphore()
        for nbr in (left, right):
            pl.semaphore_signal(
                barrier_sem, inc=1,
                device_id=(nbr,), device_id_type=pl.DeviceIdType.MESH,
            )
        pl.semaphore_wait(barrier_sem, 2)

        xb_ref[...] = x_ref[...].astype(jnp.bfloat16)
        wb_ref[...] = w_ref[...].astype(jnp.bfloat16)

        def partial_half(c, lo):
            xr = xb_ref[pl.ds(c * CHUNK, CHUNK), :]
            return jnp.dot(xr, wb_ref[:, lo:lo + HALF],
                           preferred_element_type=jnp.float32)

        NB = 8
        QTR = HALF // 2

        def rs_buf(s, dir_):
            if dir_ == 0:
                return comm_rb if s < NB else comm_r
            return comm_lb if s < NB else comm_l

        def rs_sub(s, dir_, sub):
            ss, rs = s % 2, (s + 1) % 2
            buf = rs_buf(s, dir_)
            ssem = send_r if dir_ == 0 else send_l
            rsem = recv_r if dir_ == 0 else recv_l
            tgt = right if dir_ == 0 else left
            lo = sub * QTR
            return pltpu.make_async_remote_copy(
                src_ref=buf.at[ss, :, lo:lo + QTR],
                dst_ref=buf.at[rs, :, lo:lo + QTR],
                send_sem=ssem.at[ss, sub], recv_sem=rsem.at[rs, sub],
                device_id=(tgt,), device_id_type=pl.DeviceIdType.MESH,
            )

        comm_rb[0] = partial_half(my, 0).astype(jnp.bfloat16)
        comm_lb[0] = partial_half(my, HALF).astype(jnp.bfloat16)
        pend_rs = []
        for dir_, sub in ((0, 0), (1, 0), (0, 1), (1, 1)):
            d = rs_sub(0, dir_, sub)
            d.start()
            pend_rs.append((d, dir_, sub))

        for s in range(N_DEV - 1):
            rs = (s + 1) % 2
            bf_send = s < NB
            bf_next = (s + 1) < NB
            c_r = (my + N_DEV - 1 - s) % N_DEV
            c_l = (my + s + 1) % N_DEV
            p = (partial_half(c_r, 0), partial_half(c_l, HALF))
            nxt = []
            for d, dir_, sub in pend_rs:
                d.wait()
                buf = rs_buf(s, dir_)
                lo = sub * QTR
                got = buf[rs, :, lo:lo + QTR]
                acc = (got.astype(jnp.float32) if bf_send else got) \
                    + p[dir_][:, lo:lo + QTR]
                dst = rs_buf(s + 1, dir_)
                if bf_next:
                    dst[rs, :, lo:lo + QTR] = acc.astype(jnp.bfloat16)
                else:
                    dst[rs, :, lo:lo + QTR] = acc
                if s < N_DEV - 2:
                    nd = rs_sub(s + 1, dir_, sub)
                    nd.start()
                    nxt.append((nd, dir_, sub))
            pend_rs = nxt

        own_r = (my + 1) % N_DEV
        own_l = (my + N_DEV - 1) % N_DEV

        local_amax = jnp.maximum(jnp.max(jnp.abs(comm_r[1])),
                                 jnp.max(jnp.abs(comm_l[1])))
        amax_buf[my] = jnp.full((8, 128), local_amax, jnp.float32)
        sends = []
        for o in range(1, N_DEV):
            tgt = (my + o) % N_DEV
            rd = pltpu.make_async_remote_copy(
                src_ref=amax_buf.at[my], dst_ref=amax_buf.at[my],
                send_sem=asend.at[o], recv_sem=arecv.at[my],
                device_id=(tgt,), device_id_type=pl.DeviceIdType.MESH,
            )
            rd.start()
            sends.append(rd)
        for o in range(1, N_DEV):
            src = (my + o) % N_DEV
            rd = pltpu.make_async_remote_copy(
                src_ref=amax_buf.at[src], dst_ref=amax_buf.at[src],
                send_sem=asend.at[o], recv_sem=arecv.at[src],
                device_id=(src,), device_id_type=pl.DeviceIdType.MESH,
            )
            rd.wait_recv()
        for rd in sends:
            rd.wait_send()
        amax = jnp.max(amax_buf[:, 0, 0])
        scale = amax / 448.0
        inv_scale = 1.0 / scale

        def ag_hop(t):
            ss, rs = t % 2, (t + 1) % 2
            r = pltpu.make_async_remote_copy(
                src_ref=q_r.at[ss], dst_ref=q_r.at[rs],
                send_sem=qsend_r.at[ss], recv_sem=qrecv_r.at[rs],
                device_id=(right,), device_id_type=pl.DeviceIdType.MESH,
            )
            l = pltpu.make_async_remote_copy(
                src_ref=q_l.at[ss], dst_ref=q_l.at[rs],
                send_sem=qsend_l.at[ss], recv_sem=qrecv_l.at[rs],
                device_id=(left,), device_id_type=pl.DeviceIdType.MESH,
            )
            r.start()
            l.start()
            return r, l

        q_r[0] = (comm_r[1] * inv_scale).astype(F8)
        q_l[0] = (comm_l[1] * inv_scale).astype(F8)
        pend = ag_hop(0)
        out_ref[pl.ds(own_r * CHUNK, CHUNK), 0:HALF] = (
            q_r[0].astype(jnp.float32) * scale)
        out_ref[pl.ds(own_l * CHUNK, CHUNK), HALF:N] = (
            q_l[0].astype(jnp.float32) * scale)
        for t in range(N_DEV - 1):
            rs = (t + 1) % 2
            pend[0].wait()
            pend[1].wait()
            if t < N_DEV - 2:
                pend = ag_hop(t + 1)
            c_r = (my + N_DEV - t) % N_DEV
            c_l = (my + t) % N_DEV
            out_ref[pl.ds(c_r * CHUNK, CHUNK), 0:HALF] = (
                q_r[rs].astype(jnp.float32) * scale)
            out_ref[pl.ds(c_l * CHUNK, CHUNK), HALF:N] = (
                q_l[rs].astype(jnp.float32) * scale)

    return pl.pallas_call(
        body,
        out_shape=jax.ShapeDtypeStruct((M, N), jnp.float32),
        in_specs=[
            pl.BlockSpec(memory_space=pltpu.VMEM),
            pl.BlockSpec(memory_space=pltpu.VMEM),
        ],
        out_specs=pl.BlockSpec(memory_space=pltpu.VMEM),
        scratch_shapes=[
            pltpu.VMEM((M, k_per), jnp.bfloat16),
            pltpu.VMEM((k_per, N), jnp.bfloat16),
            pltpu.VMEM((2, CHUNK, HALF), jnp.float32),
            pltpu.VMEM((2, CHUNK, HALF), jnp.float32),
            pltpu.VMEM((2, CHUNK, HALF), jnp.bfloat16),
            pltpu.VMEM((2, CHUNK, HALF), jnp.bfloat16),
            pltpu.SemaphoreType.DMA((2, 2)),
            pltpu.SemaphoreType.DMA((2, 2)),
            pltpu.SemaphoreType.DMA((2, 2)),
            pltpu.SemaphoreType.DMA((2, 2)),
            pltpu.VMEM((2, CHUNK, HALF), F8),
            pltpu.VMEM((2, CHUNK, HALF), F8),
            pltpu.SemaphoreType.DMA((2,)),
            pltpu.SemaphoreType.DMA((2,)),
            pltpu.SemaphoreType.DMA((2,)),
            pltpu.SemaphoreType.DMA((2,)),
            pltpu.VMEM((N_DEV, 8, 128), jnp.float32),
            pltpu.SemaphoreType.DMA((N_DEV,)),
            pltpu.SemaphoreType.DMA((N_DEV,)),
        ],
        compiler_params=pltpu.CompilerParams(
            collective_id=7, vmem_limit_bytes=100 * 1024 * 1024
        ),
    )(x, w_mat)


# device time: 236524 ns/iter; 3.3138x vs baseline; 1.1089x over previous
import jax
import jax.numpy as jnp
from jax import lax
from jax.experimental import pallas as pl
from jax.experimental.pallas import tpu as pltpu

N_DEV = 16
M = 4096
N = 2048
CHUNK = M // N_DEV
HALF = N // 2

F8 = jnp.float8_e4m3fn


def kernel(x, w_mat):
    k_per = x.shape[1]
    assert x.shape == (M, k_per), x.shape
    assert w_mat.shape == (k_per, N), w_mat.shape

    def body(x_ref, w_ref, out_ref, xb_ref, wb_ref,
             comm_r, comm_l, comm_rb, comm_lb,
             send_r, recv_r, send_l, recv_l,
             q_r, q_l, qsend_r, qrecv_r, qsend_l, qrecv_l,
             amax_buf, asend, arecv):
        my = lax.axis_index("i")
        left = (my + N_DEV - 1) % N_DEV
        right = (my + 1) % N_DEV

        barrier_sem = pltpu.get_barrier_semaphore()
        for nbr in (left, right):
            pl.semaphore_signal(
                barrier_sem, inc=1,
                device_id=(nbr,), device_id_type=pl.DeviceIdType.MESH,
            )
        pl.semaphore_wait(barrier_sem, 2)

        xb_ref[...] = x_ref[...].astype(jnp.bfloat16)
        wb_ref[...] = w_ref[...].astype(jnp.bfloat16)

        def partial_half(c, lo):
            xr = xb_ref[pl.ds(c * CHUNK, CHUNK), :]
            return jnp.dot(xr, wb_ref[:, lo:lo + HALF],
                           preferred_element_type=jnp.float32)

        NB = 10
        QTR = HALF // 2

        def rs_buf(s, dir_):
            if dir_ == 0:
                return comm_rb if s < NB else comm_r
            return comm_lb if s < NB else comm_l

        def rs_sub(s, dir_, sub):
            ss, rs = s % 2, (s + 1) % 2
            buf = rs_buf(s, dir_)
            ssem = send_r if dir_ == 0 else send_l
            rsem = recv_r if dir_ == 0 else recv_l
            tgt = right if dir_ == 0 else left
            lo = sub * QTR
            return pltpu.make_async_remote_copy(
                src_ref=buf.at[ss, :, lo:lo + QTR],
                dst_ref=buf.at[rs, :, lo:lo + QTR],
                send_sem=ssem.at[ss, sub], recv_sem=rsem.at[rs, sub],
                device_id=(tgt,), device_id_type=pl.DeviceIdType.MESH,
            )

        comm_rb[0] = partial_half(my, 0).astype(jnp.bfloat16)
        comm_lb[0] = partial_half(my, HALF).astype(jnp.bfloat16)
        pend_rs = []
        for dir_, sub in ((0, 0), (1, 0), (0, 1), (1, 1)):
            d = rs_sub(0, dir_, sub)
            d.start()
            pend_rs.append((d, dir_, sub))

        for s in range(N_DEV - 1):
            rs = (s + 1) % 2
            bf_send = s < NB
            bf_next = (s + 1) < NB
            c_r = (my + N_DEV - 1 - s) % N_DEV
            c_l = (my + s + 1) % N_DEV
            p = (partial_half(c_r, 0), partial_half(c_l, HALF))
            nxt = []
            for d, dir_, sub in pend_rs:
                d.wait()
                buf = rs_buf(s, dir_)
                lo = sub * QTR
                got = buf[rs, :, lo:lo + QTR]
                acc = (got.astype(jnp.float32) if bf_send else got) \
                    + p[dir_][:, lo:lo + QTR]
                dst = rs_buf(s + 1, dir_)
                if bf_next:
                    dst[rs, :, lo:lo + QTR] = acc.astype(jnp.bfloat16)
                else:
                    dst[rs, :, lo:lo + QTR] = acc
                if s < N_DEV - 2:
                    nd = rs_sub(s + 1, dir_, sub)
                    nd.start()
                    nxt.append((nd, dir_, sub))
            pend_rs = nxt

        own_r = (my + 1) % N_DEV
        own_l = (my + N_DEV - 1) % N_DEV

        local_amax = jnp.maximum(jnp.max(jnp.abs(comm_r[1])),
                                 jnp.max(jnp.abs(comm_l[1])))
        amax_buf[my] = jnp.full((8, 128), local_amax, jnp.float32)
        sends = []
        for o in range(1, N_DEV):
            tgt = (my + o) % N_DEV
            rd = pltpu.make_async_remote_copy(
                src_ref=amax_buf.at[my], dst_ref=amax_buf.at[my],
                send_sem=asend.at[o], recv_sem=arecv.at[my],
                device_id=(tgt,), device_id_type=pl.DeviceIdType.MESH,
            )
            rd.start()
            sends.append(rd)
        for o in range(1, N_DEV):
            src = (my + o) % N_DEV
            rd = pltpu.make_async_remote_copy(
                src_ref=amax_buf.at[src], dst_ref=amax_buf.at[src],
                send_sem=asend.at[o], recv_sem=arecv.at[src],
                device_id=(src,), device_id_type=pl.DeviceIdType.MESH,
            )
            rd.wait_recv()
        for rd in sends:
            rd.wait_send()
        amax = jnp.max(amax_buf[:, 0, 0])
        scale = amax / 448.0
        inv_scale = 1.0 / scale

        def ag_hop(t):
            ss, rs = t % 2, (t + 1) % 2
            r = pltpu.make_async_remote_copy(
                src_ref=q_r.at[ss], dst_ref=q_r.at[rs],
                send_sem=qsend_r.at[ss], recv_sem=qrecv_r.at[rs],
                device_id=(right,), device_id_type=pl.DeviceIdType.MESH,
            )
            l = pltpu.make_async_remote_copy(
                src_ref=q_l.at[ss], dst_ref=q_l.at[rs],
                send_sem=qsend_l.at[ss], recv_sem=qrecv_l.at[rs],
                device_id=(left,), device_id_type=pl.DeviceIdType.MESH,
            )
            r.start()
            l.start()
            return r, l

        q_r[0] = (comm_r[1] * inv_scale).astype(F8)
        q_l[0] = (comm_l[1] * inv_scale).astype(F8)
        pend = ag_hop(0)
        out_ref[pl.ds(own_r * CHUNK, CHUNK), 0:HALF] = (
            q_r[0].astype(jnp.float32) * scale).astype(jnp.bfloat16)
        out_ref[pl.ds(own_l * CHUNK, CHUNK), HALF:N] = (
            q_l[0].astype(jnp.float32) * scale).astype(jnp.bfloat16)
        for t in range(N_DEV - 1):
            rs = (t + 1) % 2
            pend[0].wait()
            pend[1].wait()
            if t < N_DEV - 2:
                pend = ag_hop(t + 1)
            c_r = (my + N_DEV - t) % N_DEV
            c_l = (my + t) % N_DEV
            out_ref[pl.ds(c_r * CHUNK, CHUNK), 0:HALF] = (
                q_r[rs].astype(jnp.float32) * scale).astype(jnp.bfloat16)
            out_ref[pl.ds(c_l * CHUNK, CHUNK), HALF:N] = (
                q_l[rs].astype(jnp.float32) * scale).astype(jnp.bfloat16)

    return pl.pallas_call(
        body,
        out_shape=jax.ShapeDtypeStruct((M, N), jnp.bfloat16),
        in_specs=[
            pl.BlockSpec(memory_space=pltpu.VMEM),
            pl.BlockSpec(memory_space=pltpu.VMEM),
        ],
        out_specs=pl.BlockSpec(memory_space=pltpu.VMEM),
        scratch_shapes=[
            pltpu.VMEM((M, k_per), jnp.bfloat16),
            pltpu.VMEM((k_per, N), jnp.bfloat16),
            pltpu.VMEM((2, CHUNK, HALF), jnp.float32),
            pltpu.VMEM((2, CHUNK, HALF), jnp.float32),
            pltpu.VMEM((2, CHUNK, HALF), jnp.bfloat16),
            pltpu.VMEM((2, CHUNK, HALF), jnp.bfloat16),
            pltpu.SemaphoreType.DMA((2, 2)),
            pltpu.SemaphoreType.DMA((2, 2)),
            pltpu.SemaphoreType.DMA((2, 2)),
            pltpu.SemaphoreType.DMA((2, 2)),
            pltpu.VMEM((2, CHUNK, HALF), F8),
            pltpu.VMEM((2, CHUNK, HALF), F8),
            pltpu.SemaphoreType.DMA((2,)),
            pltpu.SemaphoreType.DMA((2,)),
            pltpu.SemaphoreType.DMA((2,)),
            pltpu.SemaphoreType.DMA((2,)),
            pltpu.VMEM((N_DEV, 8, 128), jnp.float32),
            pltpu.SemaphoreType.DMA((N_DEV,)),
            pltpu.SemaphoreType.DMA((N_DEV,)),
        ],
        compiler_params=pltpu.CompilerParams(
            collective_id=7, vmem_limit_bytes=100 * 1024 * 1024
        ),
    )(x, w_mat)


# device time: 220159 ns/iter; 3.5601x vs baseline; 1.0743x over previous
import jax
import jax.numpy as jnp
from jax import lax
from jax.experimental import pallas as pl
from jax.experimental.pallas import tpu as pltpu

N_DEV = 16
M = 4096
N = 2048
CHUNK = M // N_DEV
HALF = N // 2

F8 = jnp.float8_e4m3fn

_NXT = [4, 0, 6, 2, 8, 1, 10, 3, 12, 5, 14, 7, 15, 9, 13, 11]
_PRV = [1, 5, 3, 7, 0, 9, 2, 11, 4, 13, 6, 15, 8, 14, 10, 12]
_ORD = [0, 15, 8, 7, 1, 14, 9, 6, 2, 13, 10, 5, 3, 12, 11, 4]


def kernel(x, w_mat):
    k_per = x.shape[1]
    assert x.shape == (M, k_per), x.shape
    assert w_mat.shape == (k_per, N), w_mat.shape

    def body(x_ref, w_ref, meta_ref, out_ref, xb_ref, wb_ref,
             comm_r, comm_l, comm_rb, comm_lb,
             send_r, recv_r, send_l, recv_l,
             q_r, q_l, qsend_r, qrecv_r, qsend_l, qrecv_l,
             amax_buf, asend, arecv):
        my = lax.axis_index("i")
        right = meta_ref[0]
        left = meta_ref[1]
        pos = meta_ref[2]

        barrier_sem = pltpu.get_barrier_semaphore()
        for nbr in (left, right):
            pl.semaphore_signal(
                barrier_sem, inc=1,
                device_id=(nbr,), device_id_type=pl.DeviceIdType.MESH,
            )
        pl.semaphore_wait(barrier_sem, 2)

        xb_ref[...] = x_ref[...].astype(jnp.bfloat16)
        wb_ref[...] = w_ref[...].astype(jnp.bfloat16)

        def partial_half(c, lo):
            xr = xb_ref[pl.ds(c * CHUNK, CHUNK), :]
            return jnp.dot(xr, wb_ref[:, lo:lo + HALF],
                           preferred_element_type=jnp.float32)

        NB = 10
        QTR = HALF // 2

        def rs_buf(s, dir_):
            if dir_ == 0:
                return comm_rb if s < NB else comm_r
            return comm_lb if s < NB else comm_l

        def rs_sub(s, dir_, sub):
            ss, rs = s % 2, (s + 1) % 2
            buf = rs_buf(s, dir_)
            ssem = send_r if dir_ == 0 else send_l
            rsem = recv_r if dir_ == 0 else recv_l
            tgt = right if dir_ == 0 else left
            lo = sub * QTR
            return pltpu.make_async_remote_copy(
                src_ref=buf.at[ss, :, lo:lo + QTR],
                dst_ref=buf.at[rs, :, lo:lo + QTR],
                send_sem=ssem.at[ss, sub], recv_sem=rsem.at[rs, sub],
                device_id=(tgt,), device_id_type=pl.DeviceIdType.MESH,
            )

        comm_rb[0] = partial_half(pos, 0).astype(jnp.bfloat16)
        comm_lb[0] = partial_half(pos, HALF).astype(jnp.bfloat16)
        pend_rs = []
        for dir_, sub in ((0, 0), (1, 0), (0, 1), (1, 1)):
            d = rs_sub(0, dir_, sub)
            d.start()
            pend_rs.append((d, dir_, sub))

        for s in range(N_DEV - 1):
            rs = (s + 1) % 2
            bf_send = s < NB
            bf_next = (s + 1) < NB
            c_r = (pos + N_DEV - 1 - s) % N_DEV
            c_l = (pos + s + 1) % N_DEV
            p = (partial_half(c_r, 0), partial_half(c_l, HALF))
            nxt = []
            for d, dir_, sub in pend_rs:
                d.wait()
                buf = rs_buf(s, dir_)
                lo = sub * QTR
                got = buf[rs, :, lo:lo + QTR]
                acc = (got.astype(jnp.float32) if bf_send else got) \
                    + p[dir_][:, lo:lo + QTR]
                dst = rs_buf(s + 1, dir_)
                if bf_next:
                    dst[rs, :, lo:lo + QTR] = acc.astype(jnp.bfloat16)
                else:
                    dst[rs, :, lo:lo + QTR] = acc
                if s < N_DEV - 2:
                    nd = rs_sub(s + 1, dir_, sub)
                    nd.start()
                    nxt.append((nd, dir_, sub))
            pend_rs = nxt

        own_r = (pos + 1) % N_DEV
        own_l = (pos + N_DEV - 1) % N_DEV

        local_amax = jnp.maximum(jnp.max(jnp.abs(comm_r[1])),
                                 jnp.max(jnp.abs(comm_l[1])))
        amax_buf[my] = jnp.full((8, 128), local_amax, jnp.float32)
        sends = []
        for o in range(1, N_DEV):
            tgt = (my + o) % N_DEV
            rd = pltpu.make_async_remote_copy(
                src_ref=amax_buf.at[my], dst_ref=amax_buf.at[my],
                send_sem=asend.at[o], recv_sem=arecv.at[my],
                device_id=(tgt,), device_id_type=pl.DeviceIdType.MESH,
            )
            rd.start()
            sends.append(rd)
        for o in range(1, N_DEV):
            src = (my + o) % N_DEV
            rd = pltpu.make_async_remote_copy(
                src_ref=amax_buf.at[src], dst_ref=amax_buf.at[src],
                send_sem=asend.at[o], recv_sem=arecv.at[src],
                device_id=(src,), device_id_type=pl.DeviceIdType.MESH,
            )
            rd.wait_recv()
        for rd in sends:
            rd.wait_send()
        amax = jnp.max(amax_buf[:, 0, 0])
        scale = amax / 448.0
        inv_scale = 1.0 / scale

        def ag_hop(t):
            ss, rs = t % 2, (t + 1) % 2
            r = pltpu.make_async_remote_copy(
                src_ref=q_r.at[ss], dst_ref=q_r.at[rs],
                send_sem=qsend_r.at[ss], recv_sem=qrecv_r.at[rs],
                device_id=(right,), device_id_type=pl.DeviceIdType.MESH,
            )
            l = pltpu.make_async_remote_copy(
                src_ref=q_l.at[ss], dst_ref=q_l.at[rs],
                send_sem=qsend_l.at[ss], recv_sem=qrecv_l.at[rs],
                device_id=(left,), device_id_type=pl.DeviceIdType.MESH,
            )
            r.start()
            l.start()
            return r, l

        q_r[0] = (comm_r[1] * inv_scale).astype(F8)
        q_l[0] = (comm_l[1] * inv_scale).astype(F8)
        pend = ag_hop(0)
        out_ref[pl.ds(own_r * CHUNK, CHUNK), 0:HALF] = (
            q_r[0].astype(jnp.float32) * scale).astype(jnp.bfloat16)
        out_ref[pl.ds(own_l * CHUNK, CHUNK), HALF:N] = (
            q_l[0].astype(jnp.float32) * scale).astype(jnp.bfloat16)
        for t in range(N_DEV - 1):
            rs = (t + 1) % 2
            pend[0].wait()
            pend[1].wait()
            if t < N_DEV - 2:
                pend = ag_hop(t + 1)
            c_r = (pos + N_DEV - t) % N_DEV
            c_l = (pos + t) % N_DEV
            out_ref[pl.ds(c_r * CHUNK, CHUNK), 0:HALF] = (
                q_r[rs].astype(jnp.float32) * scale).astype(jnp.bfloat16)
            out_ref[pl.ds(c_l * CHUNK, CHUNK), HALF:N] = (
                q_l[rs].astype(jnp.float32) * scale).astype(jnp.bfloat16)

    idx = lax.axis_index("i")
    meta = jnp.stack([
        jnp.asarray(_NXT, jnp.int32)[idx],
        jnp.asarray(_PRV, jnp.int32)[idx],
        jnp.asarray(_ORD, jnp.int32)[idx],
    ])

    return pl.pallas_call(
        body,
        out_shape=jax.ShapeDtypeStruct((M, N), jnp.bfloat16),
        in_specs=[
            pl.BlockSpec(memory_space=pltpu.VMEM),
            pl.BlockSpec(memory_space=pltpu.VMEM),
            pl.BlockSpec(memory_space=pltpu.SMEM),
        ],
        out_specs=pl.BlockSpec(memory_space=pltpu.VMEM),
        scratch_shapes=[
            pltpu.VMEM((M, k_per), jnp.bfloat16),
            pltpu.VMEM((k_per, N), jnp.bfloat16),
            pltpu.VMEM((2, CHUNK, HALF), jnp.float32),
            pltpu.VMEM((2, CHUNK, HALF), jnp.float32),
            pltpu.VMEM((2, CHUNK, HALF), jnp.bfloat16),
            pltpu.VMEM((2, CHUNK, HALF), jnp.bfloat16),
            pltpu.SemaphoreType.DMA((2, 2)),
            pltpu.SemaphoreType.DMA((2, 2)),
            pltpu.SemaphoreType.DMA((2, 2)),
            pltpu.SemaphoreType.DMA((2, 2)),
            pltpu.VMEM((2, CHUNK, HALF), F8),
            pltpu.VMEM((2, CHUNK, HALF), F8),
            pltpu.SemaphoreType.DMA((2,)),
            pltpu.SemaphoreType.DMA((2,)),
            pltpu.SemaphoreType.DMA((2,)),
            pltpu.SemaphoreType.DMA((2,)),
            pltpu.VMEM((N_DEV, 8, 128), jnp.float32),
            pltpu.SemaphoreType.DMA((N_DEV,)),
            pltpu.SemaphoreType.DMA((N_DEV,)),
        ],
        compiler_params=pltpu.CompilerParams(
            collective_id=7, vmem_limit_bytes=100 * 1024 * 1024
        ),
    )(x, w_mat, meta)


# device time: 200449 ns/iter; 3.9101x vs baseline; 1.0983x over previous
import jax
import jax.numpy as jnp
from jax import lax
from jax.experimental import pallas as pl
from jax.experimental.pallas import tpu as pltpu

N_DEV = 16
M = 4096
N = 2048
CHUNK = M // N_DEV
HALF = N // 2

F8 = jnp.float8_e4m3fn

_NXT = [4, 0, 6, 2, 8, 1, 10, 3, 12, 5, 14, 7, 15, 9, 13, 11]
_PRV = [1, 5, 3, 7, 0, 9, 2, 11, 4, 13, 6, 15, 8, 14, 10, 12]
_ORD = [0, 15, 8, 7, 1, 14, 9, 6, 2, 13, 10, 5, 3, 12, 11, 4]


def kernel(x, w_mat):
    k_per = x.shape[1]
    assert x.shape == (M, k_per), x.shape
    assert w_mat.shape == (k_per, N), w_mat.shape

    def body(x_ref, w_ref, meta_ref, out_ref, xb_ref, wb_ref,
             comm_r, comm_l, comm_rb, comm_lb,
             send_r, recv_r, send_l, recv_l,
             q_r, q_l, qsend_r, qrecv_r, qsend_l, qrecv_l,
             amax_buf, asend, arecv):
        my = lax.axis_index("i")
        right = meta_ref[0]
        left = meta_ref[1]
        pos = meta_ref[2]

        barrier_sem = pltpu.get_barrier_semaphore()
        for nbr in (left, right):
            pl.semaphore_signal(
                barrier_sem, inc=1,
                device_id=(nbr,), device_id_type=pl.DeviceIdType.MESH,
            )
        pl.semaphore_wait(barrier_sem, 2)

        xb_ref[...] = x_ref[...].astype(jnp.bfloat16)
        wb_ref[...] = w_ref[...].astype(jnp.bfloat16)

        def partial_half(c, lo):
            xr = xb_ref[pl.ds(c * CHUNK, CHUNK), :]
            return jnp.dot(xr, wb_ref[:, lo:lo + HALF],
                           preferred_element_type=jnp.float32)

        NB = 10
        QTR = HALF // 2

        def rs_buf(s, dir_):
            if dir_ == 0:
                return comm_rb if s < NB else comm_r
            return comm_lb if s < NB else comm_l

        def rs_sub(s, dir_, sub):
            ss, rs = s % 2, (s + 1) % 2
            buf = rs_buf(s, dir_)
            ssem = send_r if dir_ == 0 else send_l
            rsem = recv_r if dir_ == 0 else recv_l
            tgt = right if dir_ == 0 else left
            lo = sub * QTR
            return pltpu.make_async_remote_copy(
                src_ref=buf.at[ss, :, lo:lo + QTR],
                dst_ref=buf.at[rs, :, lo:lo + QTR],
                send_sem=ssem.at[ss, sub], recv_sem=rsem.at[rs, sub],
                device_id=(tgt,), device_id_type=pl.DeviceIdType.MESH,
            )

        comm_rb[0] = partial_half(pos, 0).astype(jnp.bfloat16)
        comm_lb[0] = partial_half(pos, HALF).astype(jnp.bfloat16)
        pend_rs = []
        for dir_, sub in ((0, 0), (1, 0), (0, 1), (1, 1)):
            d = rs_sub(0, dir_, sub)
            d.start()
            pend_rs.append((d, dir_, sub))

        for s in range(N_DEV - 1):
            rs = (s + 1) % 2
            bf_send = s < NB
            bf_next = (s + 1) < NB
            c_r = (pos + N_DEV - 1 - s) % N_DEV
            c_l = (pos + s + 1) % N_DEV
            p = (partial_half(c_r, 0), partial_half(c_l, HALF))
            nxt = []
            for d, dir_, sub in pend_rs:
                d.wait()
                buf = rs_buf(s, dir_)
                lo = sub * QTR
                got = buf[rs, :, lo:lo + QTR]
                acc = (got.astype(jnp.float32) if bf_send else got) \
                    + p[dir_][:, lo:lo + QTR]
                dst = rs_buf(s + 1, dir_)
                if bf_next:
                    dst[rs, :, lo:lo + QTR] = acc.astype(jnp.bfloat16)
                else:
                    dst[rs, :, lo:lo + QTR] = acc
                if s < N_DEV - 2:
                    nd = rs_sub(s + 1, dir_, sub)
                    nd.start()
                    nxt.append((nd, dir_, sub))
            pend_rs = nxt

        own_r = (pos + 1) % N_DEV
        own_l = (pos + N_DEV - 1) % N_DEV

        local_amax = jnp.maximum(jnp.max(jnp.abs(comm_r[1])),
                                 jnp.max(jnp.abs(comm_l[1])))
        amax_buf[my] = jnp.full((8, 128), local_amax, jnp.float32)
        sends = []
        for o in range(1, N_DEV):
            tgt = (my + o) % N_DEV
            rd = pltpu.make_async_remote_copy(
                src_ref=amax_buf.at[my], dst_ref=amax_buf.at[my],
                send_sem=asend.at[o], recv_sem=arecv.at[my],
                device_id=(tgt,), device_id_type=pl.DeviceIdType.MESH,
            )
            rd.start()
            sends.append(rd)
        for o in range(1, N_DEV):
            src = (my + o) % N_DEV
            rd = pltpu.make_async_remote_copy(
                src_ref=amax_buf.at[src], dst_ref=amax_buf.at[src],
                send_sem=asend.at[o], recv_sem=arecv.at[src],
                device_id=(src,), device_id_type=pl.DeviceIdType.MESH,
            )
            rd.wait_recv()
        for rd in sends:
            rd.wait_send()
        amax = jnp.max(amax_buf[:, 0, 0])
        scale = amax / 448.0
        inv_scale = 1.0 / scale

        def ag_sub(t, dir_, sub):
            ss, rs = t % 2, (t + 1) % 2
            buf = q_r if dir_ == 0 else q_l
            ssem = qsend_r if dir_ == 0 else qsend_l
            rsem = qrecv_r if dir_ == 0 else qrecv_l
            tgt = right if dir_ == 0 else left
            lo = sub * QTR
            return pltpu.make_async_remote_copy(
                src_ref=buf.at[ss, :, lo:lo + QTR],
                dst_ref=buf.at[rs, :, lo:lo + QTR],
                send_sem=ssem.at[ss, sub], recv_sem=rsem.at[rs, sub],
                device_id=(tgt,), device_id_type=pl.DeviceIdType.MESH,
            )

        q_r[0] = (comm_r[1] * inv_scale).astype(F8)
        q_l[0] = (comm_l[1] * inv_scale).astype(F8)
        pend_ag = []
        for dir_, sub in ((0, 0), (1, 0), (0, 1), (1, 1)):
            d = ag_sub(0, dir_, sub)
            d.start()
            pend_ag.append((d, dir_, sub))
        out_ref[pl.ds(own_r * CHUNK, CHUNK), 0:HALF] = (
            q_r[0].astype(jnp.float32) * scale).astype(jnp.bfloat16)
        out_ref[pl.ds(own_l * CHUNK, CHUNK), HALF:N] = (
            q_l[0].astype(jnp.float32) * scale).astype(jnp.bfloat16)
        for t in range(N_DEV - 1):
            rs = (t + 1) % 2
            c_r = (pos + N_DEV - t) % N_DEV
            c_l = (pos + t) % N_DEV
            nxt = []
            for d, dir_, sub in pend_ag:
                d.wait()
                if t < N_DEV - 2:
                    nd = ag_sub(t + 1, dir_, sub)
                    nd.start()
                    nxt.append((nd, dir_, sub))
                lo = sub * QTR
                if dir_ == 0:
                    out_ref[pl.ds(c_r * CHUNK, CHUNK), lo:lo + QTR] = (
                        q_r[rs, :, lo:lo + QTR].astype(jnp.float32)
                        * scale).astype(jnp.bfloat16)
                else:
                    out_ref[pl.ds(c_l * CHUNK, CHUNK),
                            HALF + lo:HALF + lo + QTR] = (
                        q_l[rs, :, lo:lo + QTR].astype(jnp.float32)
                        * scale).astype(jnp.bfloat16)
            pend_ag = nxt

    idx = lax.axis_index("i")
    meta = jnp.stack([
        jnp.asarray(_NXT, jnp.int32)[idx],
        jnp.asarray(_PRV, jnp.int32)[idx],
        jnp.asarray(_ORD, jnp.int32)[idx],
    ])

    return pl.pallas_call(
        body,
        out_shape=jax.ShapeDtypeStruct((M, N), jnp.bfloat16),
        in_specs=[
            pl.BlockSpec(memory_space=pltpu.VMEM),
            pl.BlockSpec(memory_space=pltpu.VMEM),
            pl.BlockSpec(memory_space=pltpu.SMEM),
        ],
        out_specs=pl.BlockSpec(memory_space=pltpu.VMEM),
        scratch_shapes=[
            pltpu.VMEM((M, k_per), jnp.bfloat16),
            pltpu.VMEM((k_per, N), jnp.bfloat16),
            pltpu.VMEM((2, CHUNK, HALF), jnp.float32),
            pltpu.VMEM((2, CHUNK, HALF), jnp.float32),
            pltpu.VMEM((2, CHUNK, HALF), jnp.bfloat16),
            pltpu.VMEM((2, CHUNK, HALF), jnp.bfloat16),
            pltpu.SemaphoreType.DMA((2, 2)),
            pltpu.SemaphoreType.DMA((2, 2)),
            pltpu.SemaphoreType.DMA((2, 2)),
            pltpu.SemaphoreType.DMA((2, 2)),
            pltpu.VMEM((2, CHUNK, HALF), F8),
            pltpu.VMEM((2, CHUNK, HALF), F8),
            pltpu.SemaphoreType.DMA((2, 2)),
            pltpu.SemaphoreType.DMA((2, 2)),
            pltpu.SemaphoreType.DMA((2, 2)),
            pltpu.SemaphoreType.DMA((2, 2)),
            pltpu.VMEM((N_DEV, 8, 128), jnp.float32),
            pltpu.SemaphoreType.DMA((N_DEV,)),
            pltpu.SemaphoreType.DMA((N_DEV,)),
        ],
        compiler_params=pltpu.CompilerParams(
            collective_id=7, vmem_limit_bytes=100 * 1024 * 1024
        ),
    )(x, w_mat, meta)


# device time: 186709 ns/iter; 4.1979x vs baseline; 1.0736x over previous
import jax
import jax.numpy as jnp
from jax import lax
from jax.experimental import pallas as pl
from jax.experimental.pallas import tpu as pltpu

N_DEV = 16
M = 4096
N = 2048
CHUNK = M // N_DEV
HALF = N // 2

F8 = jnp.float8_e4m3fn

_NXT = [4, 0, 6, 2, 8, 1, 10, 3, 12, 5, 14, 7, 15, 9, 13, 11]
_PRV = [1, 5, 3, 7, 0, 9, 2, 11, 4, 13, 6, 15, 8, 14, 10, 12]
_ORD = [0, 15, 8, 7, 1, 14, 9, 6, 2, 13, 10, 5, 3, 12, 11, 4]


def kernel(x, w_mat):
    k_per = x.shape[1]
    assert x.shape == (M, k_per), x.shape
    assert w_mat.shape == (k_per, N), w_mat.shape

    def body(x_ref, w_ref, meta_ref, out_ref, xb_ref, wb_ref,
             comm_r, comm_l, comm_rb, comm_lb, comm_re, comm_le,
             send_r, recv_r, send_l, recv_l,
             esend_r, erecv_r, esend_l, erecv_l,
             q_r, q_l, qsend_r, qrecv_r, qsend_l, qrecv_l,
             amax_buf, asend, arecv):
        my = lax.axis_index("i")
        right = meta_ref[0]
        left = meta_ref[1]
        pos = meta_ref[2]

        barrier_sem = pltpu.get_barrier_semaphore()
        for nbr in (left, right):
            pl.semaphore_signal(
                barrier_sem, inc=1,
                device_id=(nbr,), device_id_type=pl.DeviceIdType.MESH,
            )
        pl.semaphore_wait(barrier_sem, 2)

        xb_ref[...] = x_ref[...].astype(jnp.bfloat16)
        wb_ref[...] = w_ref[...].astype(jnp.bfloat16)

        def partial_half(c, lo):
            xr = xb_ref[pl.ds(c * CHUNK, CHUNK), :]
            return jnp.dot(xr, wb_ref[:, lo:lo + HALF],
                           preferred_element_type=jnp.float32)

        NB = 10
        QTR = HALF // 2
        R512 = jnp.float32(512.0)
        INV512 = jnp.float32(1.0 / 512.0)

        def rs_descs(s, dir_, sub):
            ss, rs = s % 2, (s + 1) % 2
            hib = comm_rb if dir_ == 0 else comm_lb
            ssem = send_r if dir_ == 0 else send_l
            rsem = recv_r if dir_ == 0 else recv_l
            tgt = right if dir_ == 0 else left
            lo = sub * QTR
            ds = [pltpu.make_async_remote_copy(
                src_ref=hib.at[ss, :, lo:lo + QTR],
                dst_ref=hib.at[rs, :, lo:lo + QTR],
                send_sem=ssem.at[ss, sub], recv_sem=rsem.at[rs, sub],
                device_id=(tgt,), device_id_type=pl.DeviceIdType.MESH,
            )]
            if s >= NB:
                lob = comm_re if dir_ == 0 else comm_le
                esem = esend_r if dir_ == 0 else esend_l
                resem = erecv_r if dir_ == 0 else erecv_l
                ds.append(pltpu.make_async_remote_copy(
                    src_ref=lob.at[ss, :, lo:lo + QTR],
                    dst_ref=lob.at[rs, :, lo:lo + QTR],
                    send_sem=esem.at[ss, sub], recv_sem=resem.at[rs, sub],
                    device_id=(tgt,), device_id_type=pl.DeviceIdType.MESH,
                ))
            return ds

        comm_rb[0] = partial_half(pos, 0).astype(jnp.bfloat16)
        comm_lb[0] = partial_half(pos, HALF).astype(jnp.bfloat16)
        pend_rs = []
        for dir_, sub in ((0, 0), (1, 0), (0, 1), (1, 1)):
            ds = rs_descs(0, dir_, sub)
            for d in ds:
                d.start()
            pend_rs.append((ds, dir_, sub))

        for s in range(N_DEV - 1):
            rs = (s + 1) % 2
            dual = s >= NB
            dual_next = (s + 1) >= NB
            c_r = (pos + N_DEV - 1 - s) % N_DEV
            c_l = (pos + s + 1) % N_DEV
            p = (partial_half(c_r, 0), partial_half(c_l, HALF))
            nxt = []
            for ds, dir_, sub in pend_rs:
                for d in ds:
                    d.wait()
                hib = comm_rb if dir_ == 0 else comm_lb
                lo = sub * QTR
                got = hib[rs, :, lo:lo + QTR].astype(jnp.float32)
                if dual:
                    lob = comm_re if dir_ == 0 else comm_le
                    got = got + lob[rs, :, lo:lo + QTR].astype(jnp.float32) \
                        * INV512
                acc = got + p[dir_][:, lo:lo + QTR]
                if s == N_DEV - 2:
                    fin = comm_r if dir_ == 0 else comm_l
                    fin[rs, :, lo:lo + QTR] = acc
                else:
                    hi = acc.astype(jnp.bfloat16)
                    hib[rs, :, lo:lo + QTR] = hi
                    if dual_next:
                        lob = comm_re if dir_ == 0 else comm_le
                        lob[rs, :, lo:lo + QTR] = (
                            (acc - hi.astype(jnp.float32)) * R512).astype(F8)
                    nds = rs_descs(s + 1, dir_, sub)
                    for nd in nds:
                        nd.start()
                    nxt.append((nds, dir_, sub))
            pend_rs = nxt

        own_r = (pos + 1) % N_DEV
        own_l = (pos + N_DEV - 1) % N_DEV

        local_amax = jnp.maximum(jnp.max(jnp.abs(comm_r[1])),
                                 jnp.max(jnp.abs(comm_l[1])))
        amax_buf[my] = jnp.full((8, 128), local_amax, jnp.float32)
        sends = []
        for o in range(1, N_DEV):
            tgt = (my + o) % N_DEV
            rd = pltpu.make_async_remote_copy(
                src_ref=amax_buf.at[my], dst_ref=amax_buf.at[my],
                send_sem=asend.at[o], recv_sem=arecv.at[my],
                device_id=(tgt,), device_id_type=pl.DeviceIdType.MESH,
            )
            rd.start()
            sends.append(rd)
        for o in range(1, N_DEV):
            src = (my + o) % N_DEV
            rd = pltpu.make_async_remote_copy(
                src_ref=amax_buf.at[src], dst_ref=amax_buf.at[src],
                send_sem=asend.at[o], recv_sem=arecv.at[src],
                device_id=(src,), device_id_type=pl.DeviceIdType.MESH,
            )
            rd.wait_recv()
        for rd in sends:
            rd.wait_send()
        amax = jnp.max(amax_buf[:, 0, 0])
        scale = amax / 448.0
        inv_scale = 1.0 / scale

        def ag_sub(t, dir_, sub):
            ss, rs = t % 2, (t + 1) % 2
            buf = q_r if dir_ == 0 else q_l
            ssem = qsend_r if dir_ == 0 else qsend_l
            rsem = qrecv_r if dir_ == 0 else qrecv_l
            tgt = right if dir_ == 0 else left
            lo = sub * QTR
            return pltpu.make_async_remote_copy(
                src_ref=buf.at[ss, :, lo:lo + QTR],
                dst_ref=buf.at[rs, :, lo:lo + QTR],
                send_sem=ssem.at[ss, sub], recv_sem=rsem.at[rs, sub],
                device_id=(tgt,), device_id_type=pl.DeviceIdType.MESH,
            )

        q_r[0] = (comm_r[1] * inv_scale).astype(F8)
        q_l[0] = (comm_l[1] * inv_scale).astype(F8)
        pend_ag = []
        for dir_, sub in ((0, 0), (1, 0), (0, 1), (1, 1)):
            d = ag_sub(0, dir_, sub)
            d.start()
            pend_ag.append((d, dir_, sub))
        out_ref[pl.ds(own_r * CHUNK, CHUNK), 0:HALF] = (
            q_r[0].astype(jnp.float32) * scale).astype(jnp.bfloat16)
        out_ref[pl.ds(own_l * CHUNK, CHUNK), HALF:N] = (
            q_l[0].astype(jnp.float32) * scale).astype(jnp.bfloat16)
        for t in range(N_DEV - 1):
            rs = (t + 1) % 2
            c_r = (pos + N_DEV - t) % N_DEV
            c_l = (pos + t) % N_DEV
            nxt = []
            for d, dir_, sub in pend_ag:
                d.wait()
                if t < N_DEV - 2:
                    nd = ag_sub(t + 1, dir_, sub)
                    nd.start()
                    nxt.append((nd, dir_, sub))
                lo = sub * QTR
                if dir_ == 0:
                    out_ref[pl.ds(c_r * CHUNK, CHUNK), lo:lo + QTR] = (
                        q_r[rs, :, lo:lo + QTR].astype(jnp.float32)
                        * scale).astype(jnp.bfloat16)
                else:
                    out_ref[pl.ds(c_l * CHUNK, CHUNK),
                            HALF + lo:HALF + lo + QTR] = (
                        q_l[rs, :, lo:lo + QTR].astype(jnp.float32)
                        * scale).astype(jnp.bfloat16)
            pend_ag = nxt

    idx = lax.axis_index("i")
    meta = jnp.stack([
        jnp.asarray(_NXT, jnp.int32)[idx],
        jnp.asarray(_PRV, jnp.int32)[idx],
        jnp.asarray(_ORD, jnp.int32)[idx],
    ])

    return pl.pallas_call(
        body,
        out_shape=jax.ShapeDtypeStruct((M, N), jnp.bfloat16),
        in_specs=[
            pl.BlockSpec(memory_space=pltpu.VMEM),
            pl.BlockSpec(memory_space=pltpu.VMEM),
            pl.BlockSpec(memory_space=pltpu.SMEM),
        ],
        out_specs=pl.BlockSpec(memory_space=pltpu.VMEM),
        scratch_shapes=[
            pltpu.VMEM((M, k_per), jnp.bfloat16),
            pltpu.VMEM((k_per, N), jnp.bfloat16),
            pltpu.VMEM((2, CHUNK, HALF), jnp.float32),
            pltpu.VMEM((2, CHUNK, HALF), jnp.float32),
            pltpu.VMEM((2, CHUNK, HALF), jnp.bfloat16),
            pltpu.VMEM((2, CHUNK, HALF), jnp.bfloat16),
            pltpu.VMEM((2, CHUNK, HALF), F8),
            pltpu.VMEM((2, CHUNK, HALF), F8),
            pltpu.SemaphoreType.DMA((2, 2)),
            pltpu.SemaphoreType.DMA((2, 2)),
            pltpu.SemaphoreType.DMA((2, 2)),
            pltpu.SemaphoreType.DMA((2, 2)),
            pltpu.SemaphoreType.DMA((2, 2)),
            pltpu.SemaphoreType.DMA((2, 2)),
            pltpu.SemaphoreType.DMA((2, 2)),
            pltpu.SemaphoreType.DMA((2, 2)),
            pltpu.VMEM((2, CHUNK, HALF), F8),
            pltpu.VMEM((2, CHUNK, HALF), F8),
            pltpu.SemaphoreType.DMA((2, 2)),
            pltpu.SemaphoreType.DMA((2, 2)),
            pltpu.SemaphoreType.DMA((2, 2)),
            pltpu.SemaphoreType.DMA((2, 2)),
            pltpu.VMEM((N_DEV, 8, 128), jnp.float32),
            pltpu.SemaphoreType.DMA((N_DEV,)),
            pltpu.SemaphoreType.DMA((N_DEV,)),
        ],
        compiler_params=pltpu.CompilerParams(
            collective_id=7, vmem_limit_bytes=100 * 1024 * 1024
        ),
    )(x, w_mat, meta)
